# baseline (speedup 1.0000x reference)
import functools

import jax
import jax.numpy as jnp
from jax import lax
from jax.experimental import pallas as pl
from jax.experimental.pallas import tpu as pltpu
from jax.experimental.pallas import tpu_sc as plsc

N_NODES = 10000
N_EDGES = 320000
D_IN = 128
D_HID = 64
D_OUT = 128

NC = 2
NS = 16
NW = NC * NS
EB = 128
NB = 80
E_P = NW * NB * EB
N_P = 10240
ROWS_PT = N_P // NS
DEGW = 8

_SC_MESH = plsc.VectorSubcoreMesh(core_axis_name="c", subcore_axis_name="s")
_SC_PARAMS = pltpu.CompilerParams(use_tc_tiling_on_sc=False)


@functools.partial(
    pl.kernel,
    out_type=jax.ShapeDtypeStruct((NC, N_P, DEGW), jnp.float32),
    mesh=_SC_MESH,
    scratch_types=[
        pltpu.VMEM((NB, EB), jnp.int32),
        pltpu.VMEM((EB, DEGW), jnp.float32),
        pltpu.VMEM_SHARED((N_P, DEGW), jnp.float32),
        pltpu.SemaphoreType.DMA,
        pltpu.SemaphoreType.DMA,
        pltpu.SemaphoreType.DMA,
        pltpu.SemaphoreType.DMA,
    ],
    compiler_params=_SC_PARAMS,
)
def _deg_kernel(er_hbm, ones_hbm, zeros_hbm, out_hbm, idx_d, ones_v, acc_sh,
                s0, s1, s2, s3):
    c = lax.axis_index("c")
    s = lax.axis_index("s")
    wid = c * NS + s
    pltpu.sync_copy(er_hbm.at[1].at[pl.ds(wid * NB, NB)], idx_d)
    pltpu.sync_copy(ones_hbm, ones_v)
    pltpu.sync_copy(zeros_hbm, acc_sh.at[pl.ds(s * ROWS_PT, ROWS_PT)])
    plsc.subcore_barrier()

    sems = (s0, s1, s2, s3)

    def body(i, carry):
        for b in range(4):
            pltpu.async_copy(
                ones_v, acc_sh.at[idx_d.at[4 * i + b]], sems[b], add=True
            )
        for b in range(4):
            pltpu.make_async_copy(
                ones_v, acc_sh.at[idx_d.at[4 * i + b]], sems[b]
            ).wait()
        return carry

    lax.fori_loop(0, NB // 4, body, 0)
    plsc.subcore_barrier()
    pltpu.sync_copy(
        acc_sh.at[pl.ds(s * ROWS_PT, ROWS_PT)],
        out_hbm.at[c].at[pl.ds(s * ROWS_PT, ROWS_PT)],
    )


NBUF = 8
OFF = 4


@functools.partial(
    pl.kernel,
    out_type=jax.ShapeDtypeStruct((NC, N_P, D_HID), jnp.float32),
    mesh=_SC_MESH,
    scratch_types=[
        pltpu.VMEM((NB, EB), jnp.int32),
        pltpu.VMEM((NB, EB), jnp.int32),
        [pltpu.VMEM((EB, D_HID), jnp.float32)] * NBUF,
        pltpu.VMEM_SHARED((N_P, D_HID), jnp.float32),
        [pltpu.SemaphoreType.DMA] * NBUF,
        [pltpu.SemaphoreType.DMA] * NBUF,
    ],
    compiler_params=_SC_PARAMS,
)
def _agg_kernel(y_hbm, er_hbm, out_hbm, idx_s, idx_d, rows, acc_sh, gsem, ssem):
    c = lax.axis_index("c")
    s = lax.axis_index("s")
    wid = c * NS + s
    pltpu.sync_copy(er_hbm.at[0].at[pl.ds(wid * NB, NB)], idx_s)
    pltpu.sync_copy(er_hbm.at[1].at[pl.ds(wid * NB, NB)], idx_d)
    pltpu.sync_copy(
        y_hbm.at[pl.ds(s * ROWS_PT, ROWS_PT)],
        acc_sh.at[pl.ds(s * ROWS_PT, ROWS_PT)],
    )
    plsc.subcore_barrier()

    for b in range(OFF):
        pltpu.async_copy(y_hbm.at[idx_s.at[b]], rows[b], gsem[b])

    NI = NB // NBUF

    def body(i, carry):
        for b in range(NBUF):
            j = NBUF * i + b
            bn = (b + OFF) % NBUF
            pltpu.make_async_copy(y_hbm.at[idx_s.at[j]], rows[b], gsem[b]).wait()
            pltpu.async_copy(rows[b], acc_sh.at[idx_d.at[j]], ssem[b], add=True)
            if b < NBUF - OFF:
                @pl.when(i > 0)
                def _():
                    pltpu.make_async_copy(
                        rows[bn], acc_sh.at[idx_d.at[j + OFF - NBUF]], ssem[bn]
                    ).wait()

                pltpu.async_copy(y_hbm.at[idx_s.at[j + OFF]], rows[bn], gsem[bn])
            else:
                @pl.when(i < NI - 1)
                def _():
                    pltpu.make_async_copy(
                        rows[bn], acc_sh.at[idx_d.at[j + OFF - NBUF]], ssem[bn]
                    ).wait()
                    pltpu.async_copy(
                        y_hbm.at[idx_s.at[j + OFF]], rows[bn], gsem[bn]
                    )
        return carry

    lax.fori_loop(0, NI, body, 0)
    for b in range(NBUF):
        j = NB - NBUF + b
        pltpu.make_async_copy(rows[b], acc_sh.at[idx_d.at[j]], ssem[b]).wait()
    plsc.subcore_barrier()
    pltpu.sync_copy(
        acc_sh.at[pl.ds(s * ROWS_PT, ROWS_PT)],
        out_hbm.at[c].at[pl.ds(s * ROWS_PT, ROWS_PT)],
    )


_RB = 512


def _dinv_block(degp_ref):
    d = degp_ref[0] + degp_ref[1]
    return lax.rsqrt(1.0 + d[:, :1])


def _mm1_body(degp_ref, x_ref, w_ref, o_ref):
    xw = jnp.dot(x_ref[...], w_ref[...], preferred_element_type=jnp.float32)
    o_ref[...] = xw * _dinv_block(degp_ref)


def _hidden_body(degp_ref, zp_ref, y1_ref, b1_ref, o_ref):
    dinv = _dinv_block(degp_ref)
    z = zp_ref[0] + zp_ref[1] - y1_ref[...]
    h = jnp.maximum(z * dinv + b1_ref[...], 0.0)
    o_ref[...] = h * dinv


def _out_body(degp_ref, zp_ref, y2_ref, w2_ref, b2_ref, o_ref):
    dinv = _dinv_block(degp_ref)
    t = (zp_ref[0] + zp_ref[1] - y2_ref[...]) * dinv
    o_ref[...] = (
        jnp.dot(t, w2_ref[...], preferred_element_type=jnp.float32) + b2_ref[...]
    )


def _degp_spec():
    return pl.BlockSpec((NC, _RB, DEGW), lambda i: (0, i, 0))


def _zp_spec(w):
    return pl.BlockSpec((NC, _RB, w), lambda i: (0, i, 0))


def _row_spec(w):
    return pl.BlockSpec((_RB, w), lambda i: (i, 0))


def _full_spec(shape):
    return pl.BlockSpec(shape, lambda i: tuple(0 for _ in shape))


_GRID = (N_P // _RB,)


def kernel(x, edge_index, W1, b1, W2, b2):
    pad_e = E_P - N_EDGES
    pad_rows = N_NODES + (jnp.arange(pad_e, dtype=jnp.int32) % (N_P - N_NODES))
    er = jnp.concatenate(
        [edge_index.astype(jnp.int32), jnp.stack([pad_rows, pad_rows])], axis=1
    ).reshape(2, NW * NB, EB)
    ones_rows = jnp.ones((EB, DEGW), jnp.float32)
    zeros_stripe = jnp.zeros((ROWS_PT, DEGW), jnp.float32)

    degp = _deg_kernel(er, ones_rows, zeros_stripe)

    y1 = pl.pallas_call(
        _mm1_body,
        grid=_GRID,
        in_specs=[_degp_spec(), _row_spec(D_IN), _full_spec((D_IN, D_HID))],
        out_specs=_row_spec(D_HID),
        out_shape=jax.ShapeDtypeStruct((N_P, D_HID), jnp.float32),
    )(degp, x, W1)

    zp1 = _agg_kernel(y1, er)

    y2 = pl.pallas_call(
        _hidden_body,
        grid=_GRID,
        in_specs=[
            _degp_spec(),
            _zp_spec(D_HID),
            _row_spec(D_HID),
            _full_spec((1, D_HID)),
        ],
        out_specs=_row_spec(D_HID),
        out_shape=jax.ShapeDtypeStruct((N_P, D_HID), jnp.float32),
    )(degp, zp1, y1, b1.reshape(1, D_HID))

    zp2 = _agg_kernel(y2, er)

    out = pl.pallas_call(
        _out_body,
        grid=_GRID,
        in_specs=[
            _degp_spec(),
            _zp_spec(D_HID),
            _row_spec(D_HID),
            _full_spec((D_HID, D_OUT)),
            _full_spec((1, D_OUT)),
        ],
        out_specs=_row_spec(D_OUT),
        out_shape=jax.ShapeDtypeStruct((N_NODES, D_OUT), jnp.float32),
    )(degp, zp2, y2, W2, b2.reshape(1, D_OUT))

    return out


# speedup vs baseline: 1.0728x; 1.0728x over previous
import functools

import jax
import jax.numpy as jnp
from jax import lax
from jax.experimental import pallas as pl
from jax.experimental.pallas import tpu as pltpu
from jax.experimental.pallas import tpu_sc as plsc

N_NODES = 10000
N_EDGES = 320000
D_IN = 128
D_HID = 64
D_OUT = 128

NC = 2
NS = 16
NW = NC * NS
EB = 128
NB = 80
E_P = NW * NB * EB
N_P = 10240
ROWS_PT = N_P // NS
DEGW = 8

_SC_MESH = plsc.VectorSubcoreMesh(core_axis_name="c", subcore_axis_name="s")
_SC_PARAMS = pltpu.CompilerParams(use_tc_tiling_on_sc=False)


@functools.partial(
    pl.kernel,
    out_type=jax.ShapeDtypeStruct((NC, N_P, DEGW), jnp.float32),
    mesh=_SC_MESH,
    scratch_types=[
        pltpu.VMEM((NB, EB), jnp.int32),
        pltpu.VMEM((EB, DEGW), jnp.float32),
        pltpu.VMEM_SHARED((N_P, DEGW), jnp.float32),
        pltpu.SemaphoreType.DMA,
        pltpu.SemaphoreType.DMA,
        pltpu.SemaphoreType.DMA,
        pltpu.SemaphoreType.DMA,
    ],
    compiler_params=_SC_PARAMS,
)
def _deg_kernel(er_hbm, ones_hbm, zeros_hbm, out_hbm, idx_d, ones_v, acc_sh,
                s0, s1, s2, s3):
    c = lax.axis_index("c")
    s = lax.axis_index("s")
    wid = c * NS + s
    pltpu.sync_copy(er_hbm.at[1].at[pl.ds(wid * NB, NB)], idx_d)
    pltpu.sync_copy(ones_hbm, ones_v)
    pltpu.sync_copy(zeros_hbm, acc_sh.at[pl.ds(s * ROWS_PT, ROWS_PT)])
    plsc.subcore_barrier()

    sems = (s0, s1, s2, s3)

    def body(i, carry):
        for b in range(4):
            pltpu.async_copy(
                ones_v, acc_sh.at[idx_d.at[4 * i + b]], sems[b], add=True
            )
        for b in range(4):
            pltpu.make_async_copy(
                ones_v, acc_sh.at[idx_d.at[4 * i + b]], sems[b]
            ).wait()
        return carry

    lax.fori_loop(0, NB // 4, body, 0)
    plsc.subcore_barrier()
    pltpu.sync_copy(
        acc_sh.at[pl.ds(s * ROWS_PT, ROWS_PT)],
        out_hbm.at[c].at[pl.ds(s * ROWS_PT, ROWS_PT)],
    )


NBUF = 8
OFF = 4


@functools.partial(
    pl.kernel,
    out_type=jax.ShapeDtypeStruct((NC, N_P, D_HID), jnp.float32),
    mesh=_SC_MESH,
    scratch_types=[
        pltpu.VMEM((NB, EB), jnp.int32),
        pltpu.VMEM((NB, EB), jnp.int32),
        [pltpu.VMEM((EB, D_HID), jnp.float32)] * NBUF,
        pltpu.VMEM_SHARED((N_P, D_HID), jnp.float32),
        [pltpu.SemaphoreType.DMA] * NBUF,
        [pltpu.SemaphoreType.DMA] * NBUF,
    ],
    compiler_params=_SC_PARAMS,
)
def _agg_kernel(y_hbm, er_hbm, out_hbm, idx_s, idx_d, rows, acc_sh, gsem, ssem):
    c = lax.axis_index("c")
    s = lax.axis_index("s")
    wid = c * NS + s
    pltpu.sync_copy(er_hbm.at[0].at[pl.ds(wid * NB, NB)], idx_s)
    pltpu.sync_copy(er_hbm.at[1].at[pl.ds(wid * NB, NB)], idx_d)
    pltpu.sync_copy(
        y_hbm.at[pl.ds(s * ROWS_PT, ROWS_PT)],
        acc_sh.at[pl.ds(s * ROWS_PT, ROWS_PT)],
    )
    plsc.subcore_barrier()

    for b in range(OFF):
        pltpu.async_copy(y_hbm.at[idx_s.at[b]], rows[b], gsem[b])

    NI = NB // NBUF

    def body(i, carry):
        for b in range(NBUF):
            j = NBUF * i + b
            bn = (b + OFF) % NBUF
            pltpu.make_async_copy(y_hbm.at[idx_s.at[j]], rows[b], gsem[b]).wait()
            pltpu.async_copy(rows[b], acc_sh.at[idx_d.at[j]], ssem[b], add=True)
            if b < NBUF - OFF:
                @pl.when(i > 0)
                def _():
                    pltpu.make_async_copy(
                        rows[bn], acc_sh.at[idx_d.at[j + OFF - NBUF]], ssem[bn]
                    ).wait()

                pltpu.async_copy(y_hbm.at[idx_s.at[j + OFF]], rows[bn], gsem[bn])
            else:
                @pl.when(i < NI - 1)
                def _():
                    pltpu.make_async_copy(
                        rows[bn], acc_sh.at[idx_d.at[j + OFF - NBUF]], ssem[bn]
                    ).wait()
                    pltpu.async_copy(
                        y_hbm.at[idx_s.at[j + OFF]], rows[bn], gsem[bn]
                    )
        return carry

    lax.fori_loop(0, NI, body, 0)
    for b in range(NBUF):
        j = NB - NBUF + b
        pltpu.make_async_copy(rows[b], acc_sh.at[idx_d.at[j]], ssem[b]).wait()
    plsc.subcore_barrier()
    pltpu.sync_copy(
        acc_sh.at[pl.ds(s * ROWS_PT, ROWS_PT)],
        out_hbm.at[c].at[pl.ds(s * ROWS_PT, ROWS_PT)],
    )


_RB = 1024


def _dinv_block(degp_ref):
    d = degp_ref[0] + degp_ref[1]
    return lax.rsqrt(1.0 + d[:, :1])


def _mm1_body(degp_ref, x_ref, w_ref, o_ref):
    xw = jnp.dot(x_ref[...], w_ref[...], preferred_element_type=jnp.float32)
    o_ref[...] = xw * _dinv_block(degp_ref)


def _hidden_body(degp_ref, zp_ref, y1_ref, b1_ref, o_ref):
    dinv = _dinv_block(degp_ref)
    z = zp_ref[0] + zp_ref[1] - y1_ref[...]
    h = jnp.maximum(z * dinv + b1_ref[...], 0.0)
    o_ref[...] = h * dinv


def _out_body(degp_ref, zp_ref, y2_ref, w2_ref, b2_ref, o_ref):
    dinv = _dinv_block(degp_ref)
    t = (zp_ref[0] + zp_ref[1] - y2_ref[...]) * dinv
    o_ref[...] = (
        jnp.dot(t, w2_ref[...], preferred_element_type=jnp.float32) + b2_ref[...]
    )


def _degp_spec():
    return pl.BlockSpec((NC, _RB, DEGW), lambda i: (0, i, 0))


def _zp_spec(w):
    return pl.BlockSpec((NC, _RB, w), lambda i: (0, i, 0))


def _row_spec(w):
    return pl.BlockSpec((_RB, w), lambda i: (i, 0))


def _full_spec(shape):
    return pl.BlockSpec(shape, lambda i: tuple(0 for _ in shape))


_GRID = (N_P // _RB,)


def kernel(x, edge_index, W1, b1, W2, b2):
    pad_e = E_P - N_EDGES
    pad_rows = N_NODES + (jnp.arange(pad_e, dtype=jnp.int32) % (N_P - N_NODES))
    er = jnp.concatenate(
        [edge_index.astype(jnp.int32), jnp.stack([pad_rows, pad_rows])], axis=1
    ).reshape(2, NW * NB, EB)
    ones_rows = jnp.ones((EB, DEGW), jnp.float32)
    zeros_stripe = jnp.zeros((ROWS_PT, DEGW), jnp.float32)

    degp = _deg_kernel(er, ones_rows, zeros_stripe)

    y1 = pl.pallas_call(
        _mm1_body,
        grid=_GRID,
        in_specs=[_degp_spec(), _row_spec(D_IN), _full_spec((D_IN, D_HID))],
        out_specs=_row_spec(D_HID),
        out_shape=jax.ShapeDtypeStruct((N_P, D_HID), jnp.float32),
    )(degp, x, W1)

    zp1 = _agg_kernel(y1, er)

    y2 = pl.pallas_call(
        _hidden_body,
        grid=_GRID,
        in_specs=[
            _degp_spec(),
            _zp_spec(D_HID),
            _row_spec(D_HID),
            _full_spec((1, D_HID)),
        ],
        out_specs=_row_spec(D_HID),
        out_shape=jax.ShapeDtypeStruct((N_P, D_HID), jnp.float32),
    )(degp, zp1, y1, b1.reshape(1, D_HID))

    zp2 = _agg_kernel(y2, er)

    out = pl.pallas_call(
        _out_body,
        grid=_GRID,
        in_specs=[
            _degp_spec(),
            _zp_spec(D_HID),
            _row_spec(D_HID),
            _full_spec((D_HID, D_OUT)),
            _full_spec((1, D_OUT)),
        ],
        out_specs=_row_spec(D_OUT),
        out_shape=jax.ShapeDtypeStruct((N_NODES, D_OUT), jnp.float32),
    )(degp, zp2, y2, W2, b2.reshape(1, D_OUT))

    return out


# speedup vs baseline: 1.0895x; 1.0155x over previous
import functools

import jax
import jax.numpy as jnp
from jax import lax
from jax.experimental import pallas as pl
from jax.experimental.pallas import tpu as pltpu
from jax.experimental.pallas import tpu_sc as plsc

N_NODES = 10000
N_EDGES = 320000
D_IN = 128
D_HID = 64
D_OUT = 128

NC = 2
NS = 16
NW = NC * NS
EB = 128
NB = 80
E_P = NW * NB * EB
N_P = 10240
ROWS_PT = N_P // NS
DEGW = 8

_SC_MESH = plsc.VectorSubcoreMesh(core_axis_name="c", subcore_axis_name="s")
_SC_PARAMS = pltpu.CompilerParams(use_tc_tiling_on_sc=False)


@functools.partial(
    pl.kernel,
    out_type=jax.ShapeDtypeStruct((NC, N_P, DEGW), jnp.float32),
    mesh=_SC_MESH,
    scratch_types=[
        pltpu.VMEM((NB, EB), jnp.int32),
        pltpu.VMEM((EB, DEGW), jnp.float32),
        pltpu.VMEM_SHARED((N_P, DEGW), jnp.float32),
        pltpu.SemaphoreType.DMA,
        pltpu.SemaphoreType.DMA,
        pltpu.SemaphoreType.DMA,
        pltpu.SemaphoreType.DMA,
    ],
    compiler_params=_SC_PARAMS,
)
def _deg_kernel(er_hbm, ones_hbm, zeros_hbm, out_hbm, idx_d, ones_v, acc_sh,
                s0, s1, s2, s3):
    c = lax.axis_index("c")
    s = lax.axis_index("s")
    wid = c * NS + s
    pltpu.sync_copy(er_hbm.at[1].at[pl.ds(wid * NB, NB)], idx_d)
    pltpu.sync_copy(ones_hbm, ones_v)
    pltpu.sync_copy(zeros_hbm, acc_sh.at[pl.ds(s * ROWS_PT, ROWS_PT)])
    plsc.subcore_barrier()

    sems = (s0, s1, s2, s3)

    def body(i, carry):
        for b in range(4):
            pltpu.async_copy(
                ones_v, acc_sh.at[idx_d.at[4 * i + b]], sems[b], add=True
            )
        for b in range(4):
            pltpu.make_async_copy(
                ones_v, acc_sh.at[idx_d.at[4 * i + b]], sems[b]
            ).wait()
        return carry

    lax.fori_loop(0, NB // 4, body, 0)
    plsc.subcore_barrier()
    pltpu.sync_copy(
        acc_sh.at[pl.ds(s * ROWS_PT, ROWS_PT)],
        out_hbm.at[c].at[pl.ds(s * ROWS_PT, ROWS_PT)],
    )


NBUF = 8
OFF = 4


@functools.partial(
    pl.kernel,
    out_type=jax.ShapeDtypeStruct((NC, N_P, D_HID), jnp.float32),
    mesh=_SC_MESH,
    scratch_types=[
        pltpu.VMEM((NB, EB), jnp.int32),
        pltpu.VMEM((NB, EB), jnp.int32),
        [pltpu.VMEM((EB, D_HID), jnp.float32)] * NBUF,
        pltpu.VMEM_SHARED((N_P, D_HID), jnp.float32),
        [pltpu.SemaphoreType.DMA] * NBUF,
        [pltpu.SemaphoreType.DMA] * NBUF,
    ],
    compiler_params=_SC_PARAMS,
)
def _agg_kernel(y_hbm, er_hbm, zeros_hbm, out_hbm, idx_s, idx_d, rows, acc_sh,
                gsem, ssem):
    c = lax.axis_index("c")
    s = lax.axis_index("s")
    wid = c * NS + s
    pltpu.sync_copy(er_hbm.at[0].at[pl.ds(wid * NB, NB)], idx_s)
    pltpu.sync_copy(er_hbm.at[1].at[pl.ds(wid * NB, NB)], idx_d)

    @pl.when(c == 0)
    def _():
        pltpu.sync_copy(
            y_hbm.at[pl.ds(s * ROWS_PT, ROWS_PT)],
            acc_sh.at[pl.ds(s * ROWS_PT, ROWS_PT)],
        )

    @pl.when(c != 0)
    def _():
        pltpu.sync_copy(zeros_hbm, acc_sh.at[pl.ds(s * ROWS_PT, ROWS_PT)])

    plsc.subcore_barrier()

    for b in range(OFF):
        pltpu.async_copy(y_hbm.at[idx_s.at[b]], rows[b], gsem[b])

    NI = NB // NBUF

    def body(i, carry):
        for b in range(NBUF):
            j = NBUF * i + b
            bn = (b + OFF) % NBUF
            pltpu.make_async_copy(y_hbm.at[idx_s.at[j]], rows[b], gsem[b]).wait()
            pltpu.async_copy(rows[b], acc_sh.at[idx_d.at[j]], ssem[b], add=True)
            if b < NBUF - OFF:
                @pl.when(i > 0)
                def _():
                    pltpu.make_async_copy(
                        rows[bn], acc_sh.at[idx_d.at[j + OFF - NBUF]], ssem[bn]
                    ).wait()

                pltpu.async_copy(y_hbm.at[idx_s.at[j + OFF]], rows[bn], gsem[bn])
            else:
                @pl.when(i < NI - 1)
                def _():
                    pltpu.make_async_copy(
                        rows[bn], acc_sh.at[idx_d.at[j + OFF - NBUF]], ssem[bn]
                    ).wait()
                    pltpu.async_copy(
                        y_hbm.at[idx_s.at[j + OFF]], rows[bn], gsem[bn]
                    )
        return carry

    lax.fori_loop(0, NI, body, 0)
    for b in range(NBUF):
        j = NB - NBUF + b
        pltpu.make_async_copy(rows[b], acc_sh.at[idx_d.at[j]], ssem[b]).wait()
    plsc.subcore_barrier()
    pltpu.sync_copy(
        acc_sh.at[pl.ds(s * ROWS_PT, ROWS_PT)],
        out_hbm.at[c].at[pl.ds(s * ROWS_PT, ROWS_PT)],
    )


_HROWS = N_P // NW


def _newton_rsqrt16(x):
    xi = lax.bitcast_convert_type(x, jnp.int32)
    yi = jnp.int32(0x5F3759DF) - (xi >> 1)
    y = lax.bitcast_convert_type(yi, jnp.float32)
    half = x * 0.5
    for _ in range(3):
        y = y * (1.5 - half * y * y)
    return y


@functools.partial(
    pl.kernel,
    out_type=jax.ShapeDtypeStruct((N_P, D_HID), jnp.float32),
    mesh=_SC_MESH,
    scratch_types=[
        pltpu.VMEM((_HROWS, D_HID), jnp.float32),
        pltpu.VMEM((_HROWS, D_HID), jnp.float32),
        pltpu.VMEM((_HROWS // 2, 2 * DEGW), jnp.float32),
        pltpu.VMEM((_HROWS // 2, 2 * DEGW), jnp.float32),
        pltpu.VMEM((D_HID,), jnp.float32),
    ],
    compiler_params=_SC_PARAMS,
)
def _hidden_kernel(zp_hbm, degv_hbm, b1_hbm, out_hbm, z0, z1, d0, d1, bv):
    c = lax.axis_index("c")
    s = lax.axis_index("s")
    wid = c * NS + s
    base = wid * _HROWS
    pltpu.sync_copy(zp_hbm.at[0].at[pl.ds(base, _HROWS)], z0)
    pltpu.sync_copy(zp_hbm.at[1].at[pl.ds(base, _HROWS)], z1)
    pltpu.sync_copy(degv_hbm.at[0].at[pl.ds(base // 2, _HROWS // 2)], d0)
    pltpu.sync_copy(degv_hbm.at[1].at[pl.ds(base // 2, _HROWS // 2)], d1)
    pltpu.sync_copy(b1_hbm, bv)

    def body(m, carry):
        pair = 1.0 + d0[m, :] + d1[m, :]
        dvp = _newton_rsqrt16(pair)
        for half in range(2):
            n = 2 * m + half
            dv = jnp.full((16,), dvp[8 * half], jnp.float32)
            for k in range(D_HID // 16):
                sl = pl.ds(16 * k, 16)
                z = z0[n, sl] + z1[n, sl]
                h = jnp.maximum(z * dv + bv[sl], 0.0)
                z0[n, sl] = h * dv
        return carry

    lax.fori_loop(0, _HROWS // 2, body, 0)
    pltpu.sync_copy(z0, out_hbm.at[pl.ds(base, _HROWS)])


_RB = 1024


def _dinv_block(degp_ref):
    d = degp_ref[0] + degp_ref[1]
    return lax.rsqrt(1.0 + d[:, :1])


def _mm1_body(degp_ref, x_ref, w_ref, o_ref):
    xw = jnp.dot(x_ref[...], w_ref[...], preferred_element_type=jnp.float32)
    o_ref[...] = xw * _dinv_block(degp_ref)


def _out_body(degp_ref, zp_ref, w2_ref, b2_ref, o_ref):
    dinv = _dinv_block(degp_ref)
    t = (zp_ref[0] + zp_ref[1]) * dinv
    o_ref[...] = (
        jnp.dot(t, w2_ref[...], preferred_element_type=jnp.float32) + b2_ref[...]
    )


def _degp_spec():
    return pl.BlockSpec((NC, _RB, DEGW), lambda i: (0, i, 0))


def _zp_spec(w):
    return pl.BlockSpec((NC, _RB, w), lambda i: (0, i, 0))


def _row_spec(w):
    return pl.BlockSpec((_RB, w), lambda i: (i, 0))


def _full_spec(shape):
    return pl.BlockSpec(shape, lambda i: tuple(0 for _ in shape))


_GRID = (N_P // _RB,)


def kernel(x, edge_index, W1, b1, W2, b2):
    pad_e = E_P - N_EDGES
    pad_rows = N_NODES + (jnp.arange(pad_e, dtype=jnp.int32) % (N_P - N_NODES))
    er = jnp.concatenate(
        [edge_index.astype(jnp.int32), jnp.stack([pad_rows, pad_rows])], axis=1
    ).reshape(2, NW * NB, EB)
    ones_rows = jnp.ones((EB, DEGW), jnp.float32)
    zeros_stripe = jnp.zeros((ROWS_PT, DEGW), jnp.float32)
    zeros_rows = jnp.zeros((ROWS_PT, D_HID), jnp.float32)

    degp = _deg_kernel(er, ones_rows, zeros_stripe)

    y1 = pl.pallas_call(
        _mm1_body,
        grid=_GRID,
        in_specs=[_degp_spec(), _row_spec(D_IN), _full_spec((D_IN, D_HID))],
        out_specs=_row_spec(D_HID),
        out_shape=jax.ShapeDtypeStruct((N_P, D_HID), jnp.float32),
    )(degp, x, W1)

    zp1 = _agg_kernel(y1, er, zeros_rows)

    y2 = _hidden_kernel(zp1, degp.reshape(NC, N_P // 2, 2 * DEGW), b1)

    zp2 = _agg_kernel(y2, er, zeros_rows)

    out = pl.pallas_call(
        _out_body,
        grid=_GRID,
        in_specs=[
            _degp_spec(),
            _zp_spec(D_HID),
            _full_spec((D_HID, D_OUT)),
            _full_spec((1, D_OUT)),
        ],
        out_specs=_row_spec(D_OUT),
        out_shape=jax.ShapeDtypeStruct((N_NODES, D_OUT), jnp.float32),
    )(degp, zp2, W2, b2.reshape(1, D_OUT))

    return out


# speedup vs baseline: 1.0927x; 1.0029x over previous
import functools

import jax
import jax.numpy as jnp
from jax import lax
from jax.experimental import pallas as pl
from jax.experimental.pallas import tpu as pltpu
from jax.experimental.pallas import tpu_sc as plsc

N_NODES = 10000
N_EDGES = 320000
D_IN = 128
D_HID = 64
D_OUT = 128

NC = 2
NS = 16
NW = NC * NS
EB = 128
NB = 80
E_P = NW * NB * EB
N_P = 10240
ROWS_PT = N_P // NS
DEGW = 8

_SC_MESH = plsc.VectorSubcoreMesh(core_axis_name="c", subcore_axis_name="s")
_SC_PARAMS = pltpu.CompilerParams(use_tc_tiling_on_sc=False)


@functools.partial(
    pl.kernel,
    out_type=jax.ShapeDtypeStruct((NC, N_P, DEGW), jnp.float32),
    mesh=_SC_MESH,
    scratch_types=[
        pltpu.VMEM((NB, EB), jnp.int32),
        pltpu.VMEM((EB, DEGW), jnp.float32),
        pltpu.VMEM_SHARED((N_P, DEGW), jnp.float32),
        pltpu.SemaphoreType.DMA,
        pltpu.SemaphoreType.DMA,
        pltpu.SemaphoreType.DMA,
        pltpu.SemaphoreType.DMA,
    ],
    compiler_params=_SC_PARAMS,
)
def _deg_kernel(er_hbm, ones_hbm, zeros_hbm, out_hbm, idx_d, ones_v, acc_sh,
                s0, s1, s2, s3):
    c = lax.axis_index("c")
    s = lax.axis_index("s")
    wid = c * NS + s
    pltpu.sync_copy(er_hbm.at[1].at[pl.ds(wid * NB, NB)], idx_d)
    pltpu.sync_copy(ones_hbm, ones_v)
    pltpu.sync_copy(zeros_hbm, acc_sh.at[pl.ds(s * ROWS_PT, ROWS_PT)])
    plsc.subcore_barrier()

    sems = (s0, s1, s2, s3)

    def body(i, carry):
        for b in range(4):
            pltpu.async_copy(
                ones_v, acc_sh.at[idx_d.at[4 * i + b]], sems[b], add=True
            )
        for b in range(4):
            pltpu.make_async_copy(
                ones_v, acc_sh.at[idx_d.at[4 * i + b]], sems[b]
            ).wait()
        return carry

    lax.fori_loop(0, NB // 4, body, 0)
    plsc.subcore_barrier()
    pltpu.sync_copy(
        acc_sh.at[pl.ds(s * ROWS_PT, ROWS_PT)],
        out_hbm.at[c].at[pl.ds(s * ROWS_PT, ROWS_PT)],
    )


NBUF = 8
OFF = 4


@functools.partial(
    pl.kernel,
    out_type=jax.ShapeDtypeStruct((NC, N_P, D_HID), jnp.float32),
    mesh=_SC_MESH,
    scratch_types=[
        pltpu.VMEM((NB, EB), jnp.int32),
        pltpu.VMEM((NB, EB), jnp.int32),
        [pltpu.VMEM((EB, D_HID), jnp.float32)] * NBUF,
        pltpu.VMEM_SHARED((N_P, D_HID), jnp.float32),
        [pltpu.SemaphoreType.DMA] * NBUF,
        [pltpu.SemaphoreType.DMA] * NBUF,
    ],
    compiler_params=_SC_PARAMS,
)
def _agg_kernel(y_hbm, er_hbm, zeros_hbm, out_hbm, idx_s, idx_d, rows, acc_sh,
                gsem, ssem):
    c = lax.axis_index("c")
    s = lax.axis_index("s")
    wid = c * NS + s
    pltpu.sync_copy(er_hbm.at[0].at[pl.ds(wid * NB, NB)], idx_s)
    pltpu.sync_copy(er_hbm.at[1].at[pl.ds(wid * NB, NB)], idx_d)

    @pl.when(c == 0)
    def _():
        pltpu.sync_copy(
            y_hbm.at[pl.ds(s * ROWS_PT, ROWS_PT)],
            acc_sh.at[pl.ds(s * ROWS_PT, ROWS_PT)],
        )

    @pl.when(c != 0)
    def _():
        pltpu.sync_copy(zeros_hbm, acc_sh.at[pl.ds(s * ROWS_PT, ROWS_PT)])

    plsc.subcore_barrier()

    for b in range(OFF):
        pltpu.async_copy(y_hbm.at[idx_s.at[b]], rows[b], gsem[b])

    NI = NB // NBUF

    def body(i, carry):
        for b in range(NBUF):
            j = NBUF * i + b
            bn = (b + OFF) % NBUF
            pltpu.make_async_copy(y_hbm.at[idx_s.at[j]], rows[b], gsem[b]).wait()
            pltpu.async_copy(rows[b], acc_sh.at[idx_d.at[j]], ssem[b], add=True)
            if b < NBUF - OFF:
                @pl.when(i > 0)
                def _():
                    pltpu.make_async_copy(
                        rows[bn], acc_sh.at[idx_d.at[j + OFF - NBUF]], ssem[bn]
                    ).wait()

                pltpu.async_copy(y_hbm.at[idx_s.at[j + OFF]], rows[bn], gsem[bn])
            else:
                @pl.when(i < NI - 1)
                def _():
                    pltpu.make_async_copy(
                        rows[bn], acc_sh.at[idx_d.at[j + OFF - NBUF]], ssem[bn]
                    ).wait()
                    pltpu.async_copy(
                        y_hbm.at[idx_s.at[j + OFF]], rows[bn], gsem[bn]
                    )
        return carry

    lax.fori_loop(0, NI, body, 0)
    for b in range(NBUF):
        j = NB - NBUF + b
        pltpu.make_async_copy(rows[b], acc_sh.at[idx_d.at[j]], ssem[b]).wait()
    plsc.subcore_barrier()
    pltpu.sync_copy(
        acc_sh.at[pl.ds(s * ROWS_PT, ROWS_PT)],
        out_hbm.at[c].at[pl.ds(s * ROWS_PT, ROWS_PT)],
    )


_HROWS = N_P // NW


def _newton_rsqrt16(x):
    xi = lax.bitcast_convert_type(x, jnp.int32)
    yi = jnp.int32(0x5F3759DF) - (xi >> 1)
    y = lax.bitcast_convert_type(yi, jnp.float32)
    half = x * 0.5
    for _ in range(3):
        y = y * (1.5 - half * y * y)
    return y


def _hidden_rows(z0, z1, bv, dvp, m):
    for half in range(2):
        n = 2 * m + half
        dv = jnp.full((16,), dvp[8 * half], jnp.float32)
        for k in range(D_HID // 16):
            sl = pl.ds(16 * k, 16)
            z = z0[n, sl] + z1[n, sl]
            h = jnp.maximum(z * dv + bv[sl], 0.0)
            z0[n, sl] = h * dv


@functools.partial(
    pl.kernel,
    out_type=jax.ShapeDtypeStruct((N_P, D_HID), jnp.float32),
    mesh=_SC_MESH,
    scratch_types=[
        pltpu.VMEM((_HROWS, D_HID), jnp.float32),
        pltpu.VMEM((_HROWS, D_HID), jnp.float32),
        pltpu.VMEM((_HROWS // 2, 2 * DEGW), jnp.float32),
        pltpu.VMEM((_HROWS // 2, 2 * DEGW), jnp.float32),
        pltpu.VMEM((D_HID,), jnp.float32),
    ],
    compiler_params=_SC_PARAMS,
)
def _hidden_kernel(zp_hbm, degv_hbm, b1_hbm, out_hbm, z0, z1, d0, d1, bv):
    c = lax.axis_index("c")
    s = lax.axis_index("s")
    wid = c * NS + s
    base = wid * _HROWS
    pltpu.sync_copy(zp_hbm.at[0].at[pl.ds(base, _HROWS)], z0)
    pltpu.sync_copy(zp_hbm.at[1].at[pl.ds(base, _HROWS)], z1)
    pltpu.sync_copy(degv_hbm.at[0].at[pl.ds(base // 2, _HROWS // 2)], d0)
    pltpu.sync_copy(degv_hbm.at[1].at[pl.ds(base // 2, _HROWS // 2)], d1)
    pltpu.sync_copy(b1_hbm, bv)

    def body(i, carry):
        m0 = 2 * i
        pa = 1.0 + d0[m0, :] + d1[m0, :]
        pb = 1.0 + d0[m0 + 1, :] + d1[m0 + 1, :]
        dva = _newton_rsqrt16(pa)
        dvb = _newton_rsqrt16(pb)
        _hidden_rows(z0, z1, bv, dva, m0)
        _hidden_rows(z0, z1, bv, dvb, m0 + 1)
        return carry

    lax.fori_loop(0, _HROWS // 4, body, 0)
    pltpu.sync_copy(z0, out_hbm.at[pl.ds(base, _HROWS)])


_RB = 1024


def _dinv_block(degp_ref):
    d = degp_ref[0] + degp_ref[1]
    return lax.rsqrt(1.0 + d[:, :1])


def _mm1_body(degp_ref, x_ref, w_ref, o_ref):
    xw = jnp.dot(x_ref[...], w_ref[...], preferred_element_type=jnp.float32)
    o_ref[...] = xw * _dinv_block(degp_ref)


def _out_body(degp_ref, zp_ref, w2_ref, b2_ref, o_ref):
    dinv = _dinv_block(degp_ref)
    t = (zp_ref[0] + zp_ref[1]) * dinv
    o_ref[...] = (
        jnp.dot(t, w2_ref[...], preferred_element_type=jnp.float32) + b2_ref[...]
    )


def _degp_spec():
    return pl.BlockSpec((NC, _RB, DEGW), lambda i: (0, i, 0))


def _zp_spec(w):
    return pl.BlockSpec((NC, _RB, w), lambda i: (0, i, 0))


def _row_spec(w):
    return pl.BlockSpec((_RB, w), lambda i: (i, 0))


def _full_spec(shape):
    return pl.BlockSpec(shape, lambda i: tuple(0 for _ in shape))


_GRID = (N_P // _RB,)


def kernel(x, edge_index, W1, b1, W2, b2):
    pad_e = E_P - N_EDGES
    pad_rows = N_NODES + (jnp.arange(pad_e, dtype=jnp.int32) % (N_P - N_NODES))
    er = jnp.concatenate(
        [edge_index.astype(jnp.int32), jnp.stack([pad_rows, pad_rows])], axis=1
    ).reshape(2, NW * NB, EB)
    ones_rows = jnp.ones((EB, DEGW), jnp.float32)
    zeros_stripe = jnp.zeros((ROWS_PT, DEGW), jnp.float32)
    zeros_rows = jnp.zeros((ROWS_PT, D_HID), jnp.float32)

    degp = _deg_kernel(er, ones_rows, zeros_stripe)

    y1 = pl.pallas_call(
        _mm1_body,
        grid=_GRID,
        in_specs=[_degp_spec(), _row_spec(D_IN), _full_spec((D_IN, D_HID))],
        out_specs=_row_spec(D_HID),
        out_shape=jax.ShapeDtypeStruct((N_P, D_HID), jnp.float32),
    )(degp, x, W1)

    zp1 = _agg_kernel(y1, er, zeros_rows)

    y2 = _hidden_kernel(zp1, degp.reshape(NC, N_P // 2, 2 * DEGW), b1)

    zp2 = _agg_kernel(y2, er, zeros_rows)

    out = pl.pallas_call(
        _out_body,
        grid=_GRID,
        in_specs=[
            _degp_spec(),
            _zp_spec(D_HID),
            _full_spec((D_HID, D_OUT)),
            _full_spec((1, D_OUT)),
        ],
        out_specs=_row_spec(D_OUT),
        out_shape=jax.ShapeDtypeStruct((N_NODES, D_OUT), jnp.float32),
    )(degp, zp2, W2, b2.reshape(1, D_OUT))

    return out
